# baseline (speedup 1.0000x reference)
import jax
import jax.numpy as jnp
from jax.experimental import pallas as pl
from jax.experimental.pallas import tpu as pltpu

_BB = 128
_Q = 8


def _bcast_kernel(pos_ref, out_ref, scratch, sems):
    scratch[...] = jnp.broadcast_to(pos_ref[...][None, :, :], scratch.shape)
    nchunks = out_ref.shape[0] // _BB

    def copy(i):
        return pltpu.make_async_copy(
            scratch, out_ref.at[pl.ds(i * _BB, _BB)], sems.at[i % _Q]
        )

    for i in range(min(_Q, nchunks)):
        copy(i).start()
    for i in range(_Q, nchunks):
        copy(i - _Q).wait()
        copy(i).start()
    for i in range(max(0, nchunks - _Q), nchunks):
        copy(i).wait()


def kernel(inputs, pos_embeddings):
    batch, seq, dims = inputs.shape
    pos = pos_embeddings[:seq, :]

    return pl.pallas_call(
        _bcast_kernel,
        in_specs=[pl.BlockSpec(memory_space=pltpu.VMEM)],
        out_specs=pl.BlockSpec(memory_space=pl.ANY),
        out_shape=jax.ShapeDtypeStruct((batch, seq, dims), pos.dtype),
        scratch_shapes=[
            pltpu.VMEM((_BB, seq, dims), pos.dtype),
            pltpu.SemaphoreType.DMA((_Q,)),
        ],
    )(pos)


# speedup vs baseline: 1.0099x; 1.0099x over previous
import jax
import jax.numpy as jnp
from jax.experimental import pallas as pl
from jax.experimental.pallas import tpu as pltpu

_BATCH_BLOCK = 64


def _tile_kernel(pos_ref, out_ref):
    out_ref[...] = jnp.broadcast_to(pos_ref[...][None, :, :], out_ref.shape)


def kernel(inputs, pos_embeddings):
    batch, seq, dims = inputs.shape
    pos = pos_embeddings[:seq, :]

    bb = _BATCH_BLOCK
    while batch % bb:
        bb //= 2
    grid = (batch // bb,)

    return pl.pallas_call(
        _tile_kernel,
        grid=grid,
        in_specs=[pl.BlockSpec((seq, dims), lambda i: (0, 0))],
        out_specs=pl.BlockSpec((bb, seq, dims), lambda i: (i, 0, 0)),
        out_shape=jax.ShapeDtypeStruct((batch, seq, dims), pos.dtype),
        compiler_params=pltpu.CompilerParams(
            dimension_semantics=("parallel",),
        ),
    )(pos)
